# Initial kernel scaffold; baseline (speedup 1.0000x reference)
import functools

import jax
import jax.numpy as jnp
from jax import lax
from jax.experimental import pallas as pl
from jax.experimental.pallas import tpu as pltpu
from jax.experimental.pallas import tpu_sc as plsc

N = 10000
E = 320000
VOCAB = 512
BAG = 16
HID = 128
OUT = 7

NW = 32
N_PAD = 10240
ROWS_W = N_PAD // NW
GRP = 128
NG_E = 79
EPW = NG_E * GRP
E_PAD = NW * EPW
NG_B = (ROWS_W * BAG) // GRP

_MESH = plsc.VectorSubcoreMesh(core_axis_name="c", subcore_axis_name="s")


def _emb_body(x_hbm, emb_hbm, out_hbm, idx_v, rows_v, out_v, sem):
    c = lax.axis_index("c")
    s = lax.axis_index("s")
    wid = s * 2 + c
    pltpu.sync_copy(x_hbm.at[wid], idx_v)

    def body(j, carry):
        pltpu.async_copy(emb_hbm.at[idx_v.at[j]], rows_v, sem).wait()
        for k in range(GRP // BAG):
            for cg in range(HID // 16):
                acc = rows_v[k * BAG, pl.ds(cg * 16, 16)]
                for r in range(1, BAG):
                    acc = acc + rows_v[k * BAG + r, pl.ds(cg * 16, 16)]
                acc = jnp.maximum(acc * (1.0 / BAG), 0.0)
                out_v[pl.ds(((j * 8 + k) * HID + cg * 16), 16)] = acc
        return carry

    lax.fori_loop(0, NG_B, body, 0)
    pltpu.sync_copy(out_v, out_hbm.at[pl.ds(wid * (ROWS_W * HID), ROWS_W * HID)])


_emb_call = pl.kernel(
    _emb_body,
    out_type=jax.ShapeDtypeStruct((N_PAD * HID,), jnp.float32),
    mesh=_MESH,
    scratch_types=[
        pltpu.VMEM((NG_B, GRP), jnp.int32),
        pltpu.VMEM((GRP, HID), jnp.float32),
        pltpu.VMEM((ROWS_W * HID,), jnp.float32),
        pltpu.SemaphoreType.DMA,
    ],
)


def _gcn_body(g_hbm, srcr_hbm, dstr_hbm, out_hbm, src_v, dst_v, rows_v,
              zero_v, acc_sh, sem):
    c = lax.axis_index("c")
    s = lax.axis_index("s")
    wid = s * 2 + c
    rows_per_tile = N_PAD // 16

    for r in range(GRP):
        for cg in range(HID // 16):
            zero_v[r, pl.ds(cg * 16, 16)] = jnp.zeros((16,), jnp.float32)
    for i in range(rows_per_tile // GRP):
        pltpu.sync_copy(zero_v, acc_sh.at[pl.ds(s * rows_per_tile + i * GRP, GRP)])
    plsc.subcore_barrier()

    pltpu.sync_copy(srcr_hbm.at[pl.ds(wid * NG_E, NG_E)], src_v)
    pltpu.sync_copy(dstr_hbm.at[pl.ds(wid * NG_E, NG_E)], dst_v)

    pltpu.async_copy(g_hbm.at[src_v.at[0]], rows_v.at[pl.ds(0, GRP)], sem)

    def body(j, carry):
        buf = (j % 2) * GRP
        nbuf = ((j + 1) % 2) * GRP
        pltpu.make_async_copy(
            g_hbm.at[src_v.at[j]], rows_v.at[pl.ds(buf, GRP)], sem).wait()

        @pl.when(j + 1 < NG_E)
        def _():
            pltpu.async_copy(
                g_hbm.at[src_v.at[j + 1]], rows_v.at[pl.ds(nbuf, GRP)], sem)

        pltpu.sync_copy(rows_v.at[pl.ds(buf, GRP)], acc_sh.at[dst_v.at[j]],
                        add=True)
        return carry

    lax.fori_loop(0, NG_E, body, 0)
    plsc.subcore_barrier()
    pltpu.sync_copy(acc_sh.at[pl.ds(s * rows_per_tile, rows_per_tile)],
                    out_hbm.at[c, pl.ds(s * rows_per_tile, rows_per_tile)])


_gcn_call = pl.kernel(
    _gcn_body,
    out_type=jax.ShapeDtypeStruct((2, N_PAD, HID), jnp.float32),
    mesh=_MESH,
    scratch_types=[
        pltpu.VMEM((NG_E, GRP), jnp.int32),
        pltpu.VMEM((NG_E, GRP), jnp.int32),
        pltpu.VMEM((2 * GRP, HID), jnp.float32),
        pltpu.VMEM((GRP, HID), jnp.float32),
        pltpu.VMEM_SHARED((N_PAD, HID), jnp.float32),
        pltpu.SemaphoreType.DMA,
    ],
)


def _mm_body(a_ref, w_ref, o_ref):
    o_ref[...] = jnp.dot(a_ref[...], w_ref[...],
                         preferred_element_type=jnp.float32)


def _mm_fused_body(p0_ref, p1_ref, w_ref, o_ref):
    h = jnp.maximum(p0_ref[0] + p1_ref[0], 0.0)
    o_ref[...] = jnp.dot(h, w_ref[...], preferred_element_type=jnp.float32)


_MM_BLK = N_PAD // 4


def _mm(h, w):
    return pl.pallas_call(
        _mm_body,
        grid=(4,),
        in_specs=[
            pl.BlockSpec((_MM_BLK, HID), lambda i: (i, 0)),
            pl.BlockSpec((HID, HID), lambda i: (0, 0)),
        ],
        out_specs=pl.BlockSpec((_MM_BLK, HID), lambda i: (i, 0)),
        out_shape=jax.ShapeDtypeStruct((N_PAD, HID), jnp.float32),
    )(h, w)


def _mm_fused(p, w):
    return pl.pallas_call(
        _mm_fused_body,
        grid=(4,),
        in_specs=[
            pl.BlockSpec((1, _MM_BLK, HID), lambda i: (0, i, 0)),
            pl.BlockSpec((1, _MM_BLK, HID), lambda i: (1, i, 0)),
            pl.BlockSpec((HID, HID), lambda i: (0, 0)),
        ],
        out_specs=pl.BlockSpec((_MM_BLK, HID), lambda i: (i, 0)),
        out_shape=jax.ShapeDtypeStruct((N_PAD, HID), jnp.float32),
    )(p, p, w)


def kernel(x, edge_index, emb, W0, W1, W2, W3, W4, lin_w):
    xp = jnp.zeros((N_PAD, BAG), jnp.int32).at[:N].set(x)
    xp = xp.reshape(NW, NG_B, GRP)

    src = edge_index[0]
    dst = edge_index[1]
    pad_ids = (jnp.arange(E_PAD - E, dtype=jnp.int32) % (N_PAD - N)) + N
    srcp = jnp.concatenate([src, pad_ids]).reshape(NW * NG_E, GRP)
    dstp = jnp.concatenate([dst, pad_ids]).reshape(NW * NG_E, GRP)

    h = _emb_call(xp, emb).reshape(N_PAD, HID)
    g = _mm(h, W0)
    for W in (W1, W2, W3, W4):
        p = _gcn_call(g, srcp, dstp)
        g = _mm_fused(p, W)
    p = _gcn_call(g, srcp, dstp)

    lin_pad = jnp.zeros((HID, 128), jnp.float32).at[:, :OUT].set(lin_w)
    out = _mm_fused(p, lin_pad)
    return out[:N, :OUT]


# speedup vs baseline: 6.0942x; 6.0942x over previous
import functools

import jax
import jax.numpy as jnp
from jax import lax
from jax.experimental import pallas as pl
from jax.experimental.pallas import tpu as pltpu
from jax.experimental.pallas import tpu_sc as plsc

N = 10000
E = 320000
VOCAB = 512
BAG = 16
HID = 128
OUT = 7

NW = 32
N_PAD = 10240
ROWS_W = N_PAD // NW
GRP = 128
EGRP = 64
CHG = 16
NCH = 10
NG_E = NCH * CHG
EPW = NG_E * EGRP
E_PAD = NW * EPW
NG_B = (ROWS_W * BAG) // GRP

_MESH = plsc.VectorSubcoreMesh(core_axis_name="c", subcore_axis_name="s")


def _emb_body(x_hbm, emb_hbm, out_hbm, idx_v, rows_v, out_v, sem):
    c = lax.axis_index("c")
    s = lax.axis_index("s")
    wid = s * 2 + c
    pltpu.sync_copy(x_hbm.at[wid], idx_v)

    def body(j, carry):
        pltpu.async_copy(emb_hbm.at[idx_v.at[j]], rows_v, sem).wait()
        for k in range(GRP // BAG):
            for cg in range(HID // 16):
                acc = rows_v[k * BAG, pl.ds(cg * 16, 16)]
                for r in range(1, BAG):
                    acc = acc + rows_v[k * BAG + r, pl.ds(cg * 16, 16)]
                acc = jnp.maximum(acc * (1.0 / BAG), 0.0)
                out_v[pl.ds(((j * 8 + k) * HID + cg * 16), 16)] = acc
        return carry

    lax.fori_loop(0, NG_B, body, 0)
    pltpu.sync_copy(out_v, out_hbm.at[pl.ds(wid * (ROWS_W * HID), ROWS_W * HID)])


_emb_call = pl.kernel(
    _emb_body,
    out_type=jax.ShapeDtypeStruct((N_PAD * HID,), jnp.float32),
    mesh=_MESH,
    scratch_types=[
        pltpu.VMEM((NG_B, GRP), jnp.int32),
        pltpu.VMEM((GRP, HID), jnp.float32),
        pltpu.VMEM((ROWS_W * HID,), jnp.float32),
        pltpu.SemaphoreType.DMA,
    ],
)


def _gcn_body(g_hbm, srcr_hbm, dstr_hbm, out_hbm, src_v, dst_v, rows_v,
              acc_sh, sem):
    c = lax.axis_index("c")
    s = lax.axis_index("s")
    wid = s * 2 + c
    rows_per_tile = N_PAD // 16

    for r in range(2 * EGRP):
        for cg in range(HID // 16):
            rows_v[r, pl.ds(cg * 16, 16)] = jnp.zeros((16,), jnp.float32)
    for i in range(rows_per_tile // (2 * EGRP)):
        pltpu.sync_copy(
            rows_v, acc_sh.at[pl.ds(s * rows_per_tile + i * 2 * EGRP, 2 * EGRP)])
    plsc.subcore_barrier()

    def chunk(cc, carry):
        pltpu.sync_copy(srcr_hbm.at[wid, pl.ds(cc * CHG, CHG)], src_v)
        pltpu.sync_copy(dstr_hbm.at[wid, pl.ds(cc * CHG, CHG)], dst_v)
        pltpu.async_copy(g_hbm.at[src_v.at[0]], rows_v.at[pl.ds(0, EGRP)], sem)

        def body(j, carry2):
            buf = (j % 2) * EGRP
            nbuf = ((j + 1) % 2) * EGRP
            pltpu.make_async_copy(
                g_hbm.at[src_v.at[j]], rows_v.at[pl.ds(buf, EGRP)], sem).wait()

            @pl.when(j + 1 < CHG)
            def _():
                pltpu.async_copy(
                    g_hbm.at[src_v.at[j + 1]], rows_v.at[pl.ds(nbuf, EGRP)],
                    sem)

            pltpu.sync_copy(rows_v.at[pl.ds(buf, EGRP)],
                            acc_sh.at[dst_v.at[j]], add=True)
            return carry2

        lax.fori_loop(0, CHG, body, 0)
        return carry

    lax.fori_loop(0, NCH, chunk, 0)
    plsc.subcore_barrier()
    pltpu.sync_copy(acc_sh.at[pl.ds(s * rows_per_tile, rows_per_tile)],
                    out_hbm.at[c, pl.ds(s * rows_per_tile, rows_per_tile)])


_gcn_call = pl.kernel(
    _gcn_body,
    out_type=jax.ShapeDtypeStruct((2, N_PAD, HID), jnp.float32),
    mesh=_MESH,
    scratch_types=[
        pltpu.VMEM((CHG, EGRP), jnp.int32),
        pltpu.VMEM((CHG, EGRP), jnp.int32),
        pltpu.VMEM((2 * EGRP, HID), jnp.float32),
        pltpu.VMEM_SHARED((N_PAD, HID), jnp.float32),
        pltpu.SemaphoreType.DMA,
    ],
)


def _mm_body(a_ref, w_ref, o_ref):
    o_ref[...] = jnp.dot(a_ref[...], w_ref[...],
                         preferred_element_type=jnp.float32)


def _mm_fused_body(p0_ref, p1_ref, w_ref, o_ref):
    h = jnp.maximum(p0_ref[0] + p1_ref[0], 0.0)
    o_ref[...] = jnp.dot(h, w_ref[...], preferred_element_type=jnp.float32)


_MM_BLK = N_PAD // 4


def _mm(h, w):
    return pl.pallas_call(
        _mm_body,
        grid=(4,),
        in_specs=[
            pl.BlockSpec((_MM_BLK, HID), lambda i: (i, 0)),
            pl.BlockSpec((HID, HID), lambda i: (0, 0)),
        ],
        out_specs=pl.BlockSpec((_MM_BLK, HID), lambda i: (i, 0)),
        out_shape=jax.ShapeDtypeStruct((N_PAD, HID), jnp.float32),
    )(h, w)


def _mm_fused(p, w):
    return pl.pallas_call(
        _mm_fused_body,
        grid=(4,),
        in_specs=[
            pl.BlockSpec((1, _MM_BLK, HID), lambda i: (0, i, 0)),
            pl.BlockSpec((1, _MM_BLK, HID), lambda i: (1, i, 0)),
            pl.BlockSpec((HID, HID), lambda i: (0, 0)),
        ],
        out_specs=pl.BlockSpec((_MM_BLK, HID), lambda i: (i, 0)),
        out_shape=jax.ShapeDtypeStruct((N_PAD, HID), jnp.float32),
    )(p, p, w)


def kernel(x, edge_index, emb, W0, W1, W2, W3, W4, lin_w):
    xp = jnp.zeros((N_PAD, BAG), jnp.int32).at[:N].set(x)
    xp = xp.reshape(NW, NG_B, GRP)

    src = edge_index[0]
    dst = edge_index[1]
    pad_ids = (jnp.arange(E_PAD - E, dtype=jnp.int32) % (N_PAD - N)) + N
    srcp = jnp.concatenate([src, pad_ids]).reshape(NW, NG_E, EGRP)
    dstp = jnp.concatenate([dst, pad_ids]).reshape(NW, NG_E, EGRP)

    h = _emb_call(xp, emb).reshape(N_PAD, HID)
    g = _mm(h, W0)
    for W in (W1, W2, W3, W4):
        p = _gcn_call(g, srcp, dstp)
        g = _mm_fused(p, W)
    p = _gcn_call(g, srcp, dstp)

    lin_pad = jnp.zeros((HID, 128), jnp.float32).at[:, :OUT].set(lin_w)
    out = _mm_fused(p, lin_pad)
    return out[:N, :OUT]


# speedup vs baseline: 6.6339x; 1.0886x over previous
import functools

import jax
import jax.numpy as jnp
from jax import lax
from jax.experimental import pallas as pl
from jax.experimental.pallas import tpu as pltpu
from jax.experimental.pallas import tpu_sc as plsc

N = 10000
E = 320000
VOCAB = 512
BAG = 16
HID = 128
OUT = 7

NW = 32
N_PAD = 10240
ROWS_W = N_PAD // NW
GRP = 128
EGRP = 64
CHG = 16
NCH = 10
NG_E = NCH * CHG
EPW = NG_E * EGRP
E_PAD = NW * EPW
NG_B = (ROWS_W * BAG) // GRP

_MESH = plsc.VectorSubcoreMesh(core_axis_name="c", subcore_axis_name="s")


def _emb_body(x_hbm, emb_hbm, out_hbm, idx_v, rows_v, red_v, sem):
    c = lax.axis_index("c")
    s = lax.axis_index("s")
    wid = s * 2 + c
    pltpu.sync_copy(x_hbm.at[wid], idx_v)

    pltpu.async_copy(emb_hbm.at[idx_v.at[0]], rows_v.at[pl.ds(0, GRP)], sem)

    def _step(j, buf, nbuf):
        pltpu.make_async_copy(
            emb_hbm.at[idx_v.at[j]], rows_v.at[pl.ds(buf, GRP)], sem).wait()

        @pl.when(j + 1 < NG_B)
        def _():
            pltpu.async_copy(
                emb_hbm.at[idx_v.at[j + 1]], rows_v.at[pl.ds(nbuf, GRP)], sem)

        for k in range(GRP // BAG):
            for cg in range(HID // 16):
                acc = rows_v[buf + k * BAG, pl.ds(cg * 16, 16)]
                for r in range(1, BAG):
                    acc = acc + rows_v[buf + k * BAG + r, pl.ds(cg * 16, 16)]
                acc = jnp.maximum(acc * (1.0 / BAG), 0.0)
                red_v[k, pl.ds(cg * 16, 16)] = acc
        pltpu.sync_copy(red_v, out_hbm.at[pl.ds(wid * ROWS_W + j * 8, 8)])

    def body(i, carry):
        _step(2 * i, 0, GRP)
        _step(2 * i + 1, GRP, 0)
        return carry

    lax.fori_loop(0, NG_B // 2, body, 0)


_emb_call = pl.kernel(
    _emb_body,
    out_type=jax.ShapeDtypeStruct((N_PAD, HID), jnp.float32),
    mesh=_MESH,
    scratch_types=[
        pltpu.VMEM((NG_B, GRP), jnp.int32),
        pltpu.VMEM((2 * GRP, HID), jnp.float32),
        pltpu.VMEM((8, HID), jnp.float32),
        pltpu.SemaphoreType.DMA,
    ],
)


def _gcn_body(g_hbm, srcr_hbm, dstr_hbm, out_hbm, src_v, dst_v, rows_v,
              acc_sh, sem):
    c = lax.axis_index("c")
    s = lax.axis_index("s")
    wid = s * 2 + c
    rows_per_tile = N_PAD // 16

    for r in range(2 * EGRP):
        for cg in range(HID // 16):
            rows_v[r, pl.ds(cg * 16, 16)] = jnp.zeros((16,), jnp.float32)
    for i in range(rows_per_tile // (2 * EGRP)):
        pltpu.sync_copy(
            rows_v, acc_sh.at[pl.ds(s * rows_per_tile + i * 2 * EGRP, 2 * EGRP)])
    plsc.subcore_barrier()

    def chunk(cc, carry):
        pltpu.sync_copy(srcr_hbm.at[wid, pl.ds(cc * CHG, CHG)], src_v)
        pltpu.sync_copy(dstr_hbm.at[wid, pl.ds(cc * CHG, CHG)], dst_v)
        pltpu.async_copy(g_hbm.at[src_v.at[0]], rows_v.at[pl.ds(0, EGRP)], sem)

        def body(j, carry2):
            buf = (j % 2) * EGRP
            nbuf = ((j + 1) % 2) * EGRP
            pltpu.make_async_copy(
                g_hbm.at[src_v.at[j]], rows_v.at[pl.ds(buf, EGRP)], sem).wait()

            @pl.when(j + 1 < CHG)
            def _():
                pltpu.async_copy(
                    g_hbm.at[src_v.at[j + 1]], rows_v.at[pl.ds(nbuf, EGRP)],
                    sem)

            pltpu.sync_copy(rows_v.at[pl.ds(buf, EGRP)],
                            acc_sh.at[dst_v.at[j]], add=True)
            return carry2

        lax.fori_loop(0, CHG, body, 0)
        return carry

    lax.fori_loop(0, NCH, chunk, 0)
    plsc.subcore_barrier()
    pltpu.sync_copy(acc_sh.at[pl.ds(s * rows_per_tile, rows_per_tile)],
                    out_hbm.at[c, pl.ds(s * rows_per_tile, rows_per_tile)])


_gcn_call = pl.kernel(
    _gcn_body,
    out_type=jax.ShapeDtypeStruct((2, N_PAD, HID), jnp.float32),
    mesh=_MESH,
    scratch_types=[
        pltpu.VMEM((CHG, EGRP), jnp.int32),
        pltpu.VMEM((CHG, EGRP), jnp.int32),
        pltpu.VMEM((2 * EGRP, HID), jnp.float32),
        pltpu.VMEM_SHARED((N_PAD, HID), jnp.float32),
        pltpu.SemaphoreType.DMA,
    ],
)


def _mm_body(a_ref, w_ref, o_ref):
    o_ref[...] = jnp.dot(a_ref[...], w_ref[...],
                         preferred_element_type=jnp.float32)


def _mm_fused_body(p0_ref, p1_ref, w_ref, o_ref):
    h = jnp.maximum(p0_ref[0] + p1_ref[0], 0.0)
    o_ref[...] = jnp.dot(h, w_ref[...], preferred_element_type=jnp.float32)


_MM_BLK = N_PAD // 4


def _mm(h, w):
    return pl.pallas_call(
        _mm_body,
        grid=(4,),
        in_specs=[
            pl.BlockSpec((_MM_BLK, HID), lambda i: (i, 0)),
            pl.BlockSpec((HID, HID), lambda i: (0, 0)),
        ],
        out_specs=pl.BlockSpec((_MM_BLK, HID), lambda i: (i, 0)),
        out_shape=jax.ShapeDtypeStruct((N_PAD, HID), jnp.float32),
    )(h, w)


def _mm_fused(p, w):
    return pl.pallas_call(
        _mm_fused_body,
        grid=(4,),
        in_specs=[
            pl.BlockSpec((1, _MM_BLK, HID), lambda i: (0, i, 0)),
            pl.BlockSpec((1, _MM_BLK, HID), lambda i: (1, i, 0)),
            pl.BlockSpec((HID, HID), lambda i: (0, 0)),
        ],
        out_specs=pl.BlockSpec((_MM_BLK, HID), lambda i: (i, 0)),
        out_shape=jax.ShapeDtypeStruct((N_PAD, HID), jnp.float32),
    )(p, p, w)


def kernel(x, edge_index, emb, W0, W1, W2, W3, W4, lin_w):
    xp = jnp.zeros((N_PAD, BAG), jnp.int32).at[:N].set(x)
    xp = xp.reshape(NW, NG_B, GRP)
    xp = xp + (jnp.arange(NW, dtype=jnp.int32) * VOCAB)[:, None, None]
    emb_rep = jnp.tile(emb, (NW, 1))

    src = edge_index[0]
    dst = edge_index[1]
    pad_ids = (jnp.arange(E_PAD - E, dtype=jnp.int32) % (N_PAD - N)) + N
    srcp = jnp.concatenate([src, pad_ids]).reshape(NW, NG_E, EGRP)
    dstp = jnp.concatenate([dst, pad_ids]).reshape(NW, NG_E, EGRP)

    h = _emb_call(xp, emb_rep)
    g = _mm(h, W0)
    for W in (W1, W2, W3, W4):
        p = _gcn_call(g, srcp, dstp)
        g = _mm_fused(p, W)
    p = _gcn_call(g, srcp, dstp)

    lin_pad = jnp.zeros((HID, 128), jnp.float32).at[:, :OUT].set(lin_w)
    out = _mm_fused(p, lin_pad)
    return out[:N, :OUT]


# speedup vs baseline: 7.7830x; 1.1732x over previous
import functools

import jax
import jax.numpy as jnp
from jax import lax
from jax.experimental import pallas as pl
from jax.experimental.pallas import tpu as pltpu
from jax.experimental.pallas import tpu_sc as plsc

N = 10000
E = 320000
VOCAB = 512
BAG = 16
HID = 128
OUT = 7

NW = 32
N_PAD = 10240
ROWS_W = N_PAD // NW
GRP = 128
EGRP = 64
CHG = 16
NCH = 10
NG_E = NCH * CHG
EPW = NG_E * EGRP
E_PAD = NW * EPW
NG_B = (ROWS_W * BAG) // GRP

_MESH = plsc.VectorSubcoreMesh(core_axis_name="c", subcore_axis_name="s")


def _emb_body(x_ref, emb_ref, w_ref, o_ref):
    blk = x_ref.shape[0]
    cnt = jnp.zeros((blk, VOCAB), jnp.float32)
    iota = lax.broadcasted_iota(jnp.int32, (blk, VOCAB), 1)
    for b in range(BAG):
        cnt = cnt + (x_ref[:, b][:, None] == iota).astype(jnp.float32)
    h = jnp.maximum(
        jnp.dot(cnt, emb_ref[...], preferred_element_type=jnp.float32)
        * (1.0 / BAG), 0.0)
    o_ref[...] = jnp.dot(h, w_ref[...], preferred_element_type=jnp.float32)


def _emb_call(x, emb, w0):
    blk = N_PAD // 8
    return pl.pallas_call(
        _emb_body,
        grid=(8,),
        in_specs=[
            pl.BlockSpec((blk, BAG), lambda i: (i, 0)),
            pl.BlockSpec((VOCAB, HID), lambda i: (0, 0)),
            pl.BlockSpec((HID, HID), lambda i: (0, 0)),
        ],
        out_specs=pl.BlockSpec((blk, HID), lambda i: (i, 0)),
        out_shape=jax.ShapeDtypeStruct((N_PAD, HID), jnp.float32),
    )(x, emb, w0)


def _gcn_body(g_hbm, srcr_hbm, dstr_hbm, out_hbm, src_v, dst_v, rows_v,
              acc_sh, sem):
    c = lax.axis_index("c")
    s = lax.axis_index("s")
    wid = s * 2 + c
    rows_per_tile = N_PAD // 16

    for r in range(2 * EGRP):
        for cg in range(HID // 16):
            rows_v[r, pl.ds(cg * 16, 16)] = jnp.zeros((16,), jnp.float32)
    for i in range(rows_per_tile // (2 * EGRP)):
        pltpu.sync_copy(
            rows_v, acc_sh.at[pl.ds(s * rows_per_tile + i * 2 * EGRP, 2 * EGRP)])
    plsc.subcore_barrier()

    def chunk(cc, carry):
        pltpu.sync_copy(srcr_hbm.at[wid, pl.ds(cc * CHG, CHG)], src_v)
        pltpu.sync_copy(dstr_hbm.at[wid, pl.ds(cc * CHG, CHG)], dst_v)
        pltpu.async_copy(g_hbm.at[src_v.at[0]], rows_v.at[pl.ds(0, EGRP)], sem)

        def body(j, carry2):
            buf = (j % 2) * EGRP
            nbuf = ((j + 1) % 2) * EGRP
            pltpu.make_async_copy(
                g_hbm.at[src_v.at[j]], rows_v.at[pl.ds(buf, EGRP)], sem).wait()

            @pl.when(j + 1 < CHG)
            def _():
                pltpu.async_copy(
                    g_hbm.at[src_v.at[j + 1]], rows_v.at[pl.ds(nbuf, EGRP)],
                    sem)

            pltpu.sync_copy(rows_v.at[pl.ds(buf, EGRP)],
                            acc_sh.at[dst_v.at[j]], add=True)
            return carry2

        lax.fori_loop(0, CHG, body, 0)
        return carry

    lax.fori_loop(0, NCH, chunk, 0)
    plsc.subcore_barrier()
    pltpu.sync_copy(acc_sh.at[pl.ds(s * rows_per_tile, rows_per_tile)],
                    out_hbm.at[c, pl.ds(s * rows_per_tile, rows_per_tile)])


_gcn_call = pl.kernel(
    _gcn_body,
    out_type=jax.ShapeDtypeStruct((2, N_PAD, HID), jnp.float32),
    mesh=_MESH,
    scratch_types=[
        pltpu.VMEM((CHG, EGRP), jnp.int32),
        pltpu.VMEM((CHG, EGRP), jnp.int32),
        pltpu.VMEM((2 * EGRP, HID), jnp.float32),
        pltpu.VMEM_SHARED((N_PAD, HID), jnp.float32),
        pltpu.SemaphoreType.DMA,
    ],
)


def _mm_body(a_ref, w_ref, o_ref):
    o_ref[...] = jnp.dot(a_ref[...], w_ref[...],
                         preferred_element_type=jnp.float32)


def _mm_fused_body(p0_ref, p1_ref, w_ref, o_ref):
    h = jnp.maximum(p0_ref[0] + p1_ref[0], 0.0)
    o_ref[...] = jnp.dot(h, w_ref[...], preferred_element_type=jnp.float32)


_MM_BLK = N_PAD // 4


def _mm(h, w):
    return pl.pallas_call(
        _mm_body,
        grid=(4,),
        in_specs=[
            pl.BlockSpec((_MM_BLK, HID), lambda i: (i, 0)),
            pl.BlockSpec((HID, HID), lambda i: (0, 0)),
        ],
        out_specs=pl.BlockSpec((_MM_BLK, HID), lambda i: (i, 0)),
        out_shape=jax.ShapeDtypeStruct((N_PAD, HID), jnp.float32),
    )(h, w)


def _mm_fused(p, w):
    return pl.pallas_call(
        _mm_fused_body,
        grid=(4,),
        in_specs=[
            pl.BlockSpec((1, _MM_BLK, HID), lambda i: (0, i, 0)),
            pl.BlockSpec((1, _MM_BLK, HID), lambda i: (1, i, 0)),
            pl.BlockSpec((HID, HID), lambda i: (0, 0)),
        ],
        out_specs=pl.BlockSpec((_MM_BLK, HID), lambda i: (i, 0)),
        out_shape=jax.ShapeDtypeStruct((N_PAD, HID), jnp.float32),
    )(p, p, w)


def kernel(x, edge_index, emb, W0, W1, W2, W3, W4, lin_w):
    xp = jnp.zeros((N_PAD, BAG), jnp.int32).at[:N].set(x)

    src = edge_index[0]
    dst = edge_index[1]
    pad_ids = (jnp.arange(E_PAD - E, dtype=jnp.int32) % (N_PAD - N)) + N
    srcp = jnp.concatenate([src, pad_ids]).reshape(NW, NG_E, EGRP)
    dstp = jnp.concatenate([dst, pad_ids]).reshape(NW, NG_E, EGRP)

    g = _emb_call(xp, emb, W0)
    for W in (W1, W2, W3, W4):
        p = _gcn_call(g, srcp, dstp)
        g = _mm_fused(p, W)
    p = _gcn_call(g, srcp, dstp)

    lin_pad = jnp.zeros((HID, 128), jnp.float32).at[:, :OUT].set(lin_w)
    out = _mm_fused(p, lin_pad)
    return out[:N, :OUT]


# speedup vs baseline: 12.1140x; 1.5565x over previous
import functools

import jax
import jax.numpy as jnp
from jax import lax
from jax.experimental import pallas as pl
from jax.experimental.pallas import tpu as pltpu
from jax.experimental.pallas import tpu_sc as plsc

N = 10000
E = 320000
VOCAB = 512
BAG = 16
HID = 128
OUT = 7

NW = 32
N_PAD = 10240
ROWS_W = N_PAD // NW
GRP = 128
EGRP = 64
CHG = 16
NCH = 10
NG_E = NCH * CHG
EPW = NG_E * EGRP
E_PAD = NW * EPW
NG_B = (ROWS_W * BAG) // GRP

_MESH = plsc.VectorSubcoreMesh(core_axis_name="c", subcore_axis_name="s")


def _emb_body(x_ref, emb_ref, w_ref, o_ref):
    blk = x_ref.shape[0]
    cnt = jnp.zeros((blk, VOCAB), jnp.float32)
    iota = lax.broadcasted_iota(jnp.int32, (blk, VOCAB), 1)
    for b in range(BAG):
        cnt = cnt + (x_ref[:, b][:, None] == iota).astype(jnp.float32)
    h = jnp.maximum(
        jnp.dot(cnt, emb_ref[...], preferred_element_type=jnp.float32)
        * (1.0 / BAG), 0.0)
    o_ref[...] = jnp.dot(h, w_ref[...], preferred_element_type=jnp.float32)


def _emb_call(x, emb, w0):
    blk = N_PAD // 8
    return pl.pallas_call(
        _emb_body,
        grid=(8,),
        in_specs=[
            pl.BlockSpec((blk, BAG), lambda i: (i, 0)),
            pl.BlockSpec((VOCAB, HID), lambda i: (0, 0)),
            pl.BlockSpec((HID, HID), lambda i: (0, 0)),
        ],
        out_specs=pl.BlockSpec((blk, HID), lambda i: (i, 0)),
        out_shape=jax.ShapeDtypeStruct((N_PAD, HID), jnp.float32),
    )(x, emb, w0)


def _gcn_body(g_hbm, srcr_hbm, dstr_hbm, out_hbm, src_v, dst_v, rows_v,
              acc_sh, sem):
    c = lax.axis_index("c")
    s = lax.axis_index("s")
    wid = s * 2 + c
    rows_per_tile = N_PAD // 16

    for r in range(2 * EGRP):
        for cg in range(HID // 16):
            rows_v[r, pl.ds(cg * 16, 16)] = jnp.zeros((16,), jnp.float32)
    for i in range(rows_per_tile // (2 * EGRP)):
        pltpu.sync_copy(
            rows_v.at[pl.ds(0, 2 * EGRP)],
            acc_sh.at[pl.ds(s * rows_per_tile + i * 2 * EGRP, 2 * EGRP)])
    plsc.subcore_barrier()

    pltpu.sync_copy(srcr_hbm.at[wid, pl.ds(0, CHG)], src_v.at[0])
    pltpu.sync_copy(dstr_hbm.at[wid, pl.ds(0, CHG)], dst_v.at[0])

    def _gather(g):
        slot = (g // CHG) % 2
        pltpu.async_copy(g_hbm.at[src_v.at[slot, g % CHG]],
                         rows_v.at[pl.ds((g % 3) * EGRP, EGRP)], sem)

    _gather(0)
    _gather(1)

    def body(g, carry):
        @pl.when(jnp.logical_and(g % CHG == 0, g + CHG < NG_E))
        def _():
            nslot = ((g // CHG) + 1) % 2
            pltpu.sync_copy(
                srcr_hbm.at[wid, pl.ds((g // CHG + 1) * CHG, CHG)],
                src_v.at[nslot])
            pltpu.sync_copy(
                dstr_hbm.at[wid, pl.ds((g // CHG + 1) * CHG, CHG)],
                dst_v.at[nslot])

        slot = (g // CHG) % 2
        buf = (g % 3) * EGRP
        pltpu.make_async_copy(
            g_hbm.at[src_v.at[slot, g % CHG]], rows_v.at[pl.ds(buf, EGRP)],
            sem).wait()

        @pl.when(g + 2 < NG_E)
        def _():
            _gather(g + 2)

        pltpu.sync_copy(rows_v.at[pl.ds(buf, EGRP)],
                        acc_sh.at[dst_v.at[slot, g % CHG]], add=True)
        return carry

    lax.fori_loop(0, NG_E, body, 0)
    plsc.subcore_barrier()
    pltpu.sync_copy(acc_sh.at[pl.ds(s * rows_per_tile, rows_per_tile)],
                    out_hbm.at[c, pl.ds(s * rows_per_tile, rows_per_tile)])


_gcn_call = pl.kernel(
    _gcn_body,
    out_type=jax.ShapeDtypeStruct((2, N_PAD, HID), jnp.float32),
    mesh=_MESH,
    scratch_types=[
        pltpu.VMEM((2, CHG, EGRP), jnp.int32),
        pltpu.VMEM((2, CHG, EGRP), jnp.int32),
        pltpu.VMEM((3 * EGRP, HID), jnp.float32),
        pltpu.VMEM_SHARED((N_PAD, HID), jnp.float32),
        pltpu.SemaphoreType.DMA,
    ],
)


def _mm_body(a_ref, w_ref, o_ref):
    o_ref[...] = jnp.dot(a_ref[...], w_ref[...],
                         preferred_element_type=jnp.float32)


def _mm_fused_body(p0_ref, p1_ref, w_ref, o_ref):
    h = jnp.maximum(p0_ref[0] + p1_ref[0], 0.0)
    o_ref[...] = jnp.dot(h, w_ref[...], preferred_element_type=jnp.float32)


_MM_BLK = N_PAD // 4


def _mm(h, w):
    return pl.pallas_call(
        _mm_body,
        grid=(4,),
        in_specs=[
            pl.BlockSpec((_MM_BLK, HID), lambda i: (i, 0)),
            pl.BlockSpec((HID, HID), lambda i: (0, 0)),
        ],
        out_specs=pl.BlockSpec((_MM_BLK, HID), lambda i: (i, 0)),
        out_shape=jax.ShapeDtypeStruct((N_PAD, HID), jnp.float32),
    )(h, w)


def _mm_fused(p, w):
    return pl.pallas_call(
        _mm_fused_body,
        grid=(4,),
        in_specs=[
            pl.BlockSpec((1, _MM_BLK, HID), lambda i: (0, i, 0)),
            pl.BlockSpec((1, _MM_BLK, HID), lambda i: (1, i, 0)),
            pl.BlockSpec((HID, HID), lambda i: (0, 0)),
        ],
        out_specs=pl.BlockSpec((_MM_BLK, HID), lambda i: (i, 0)),
        out_shape=jax.ShapeDtypeStruct((N_PAD, HID), jnp.float32),
    )(p, p, w)


def kernel(x, edge_index, emb, W0, W1, W2, W3, W4, lin_w):
    xp = jnp.zeros((N_PAD, BAG), jnp.int32).at[:N].set(x)

    src = edge_index[0]
    dst = edge_index[1]
    pad_ids = (jnp.arange(E_PAD - E, dtype=jnp.int32) % (N_PAD - N)) + N
    srcp = jnp.concatenate([src, pad_ids]).reshape(NW, NG_E, EGRP)
    dstp = jnp.concatenate([dst, pad_ids]).reshape(NW, NG_E, EGRP)

    g = _emb_call(xp, emb, W0)
    for W in (W1, W2, W3, W4):
        p = _gcn_call(g, srcp, dstp)
        g = _mm_fused(p, W)
    p = _gcn_call(g, srcp, dstp)

    lin_pad = jnp.zeros((HID, 128), jnp.float32).at[:, :OUT].set(lin_w)
    out = _mm_fused(p, lin_pad)
    return out[:N, :OUT]


# speedup vs baseline: 12.6470x; 1.0440x over previous
import functools

import jax
import jax.numpy as jnp
from jax import lax
from jax.experimental import pallas as pl
from jax.experimental.pallas import tpu as pltpu
from jax.experimental.pallas import tpu_sc as plsc

N = 10000
E = 320000
VOCAB = 512
BAG = 16
HID = 128
OUT = 7

NW = 32
N_PAD = 10240
ROWS_W = N_PAD // NW
GRP = 128
EGRP = 64
CHG = 8
NCH = 20
NG_E = NCH * CHG
EPW = NG_E * EGRP
E_PAD = NW * EPW
NG_B = (ROWS_W * BAG) // GRP

_MESH = plsc.VectorSubcoreMesh(core_axis_name="c", subcore_axis_name="s")


def _emb_body(x_ref, emb_ref, w_ref, o_ref):
    blk = x_ref.shape[0]
    cnt = jnp.zeros((blk, VOCAB), jnp.float32)
    iota = lax.broadcasted_iota(jnp.int32, (blk, VOCAB), 1)
    for b in range(BAG):
        cnt = cnt + (x_ref[:, b][:, None] == iota).astype(jnp.float32)
    h = jnp.maximum(
        jnp.dot(cnt, emb_ref[...], preferred_element_type=jnp.float32)
        * (1.0 / BAG), 0.0)
    o_ref[...] = jnp.dot(h, w_ref[...], preferred_element_type=jnp.float32)


def _emb_call(x, emb, w0):
    blk = N_PAD // 8
    return pl.pallas_call(
        _emb_body,
        grid=(8,),
        in_specs=[
            pl.BlockSpec((blk, BAG), lambda i: (i, 0)),
            pl.BlockSpec((VOCAB, HID), lambda i: (0, 0)),
            pl.BlockSpec((HID, HID), lambda i: (0, 0)),
        ],
        out_specs=pl.BlockSpec((blk, HID), lambda i: (i, 0)),
        out_shape=jax.ShapeDtypeStruct((N_PAD, HID), jnp.float32),
    )(x, emb, w0)


def _gcn_body(g_hbm, srcr_hbm, dstr_hbm, out_hbm, src_v, dst_v, rows_v,
              acc_sh, sem):
    c = lax.axis_index("c")
    s = lax.axis_index("s")
    wid = s * 2 + c
    rows_per_tile = N_PAD // 16

    for r in range(2 * EGRP):
        for cg in range(HID // 16):
            rows_v[r, pl.ds(cg * 16, 16)] = jnp.zeros((16,), jnp.float32)
    for i in range(rows_per_tile // (2 * EGRP)):
        pltpu.sync_copy(
            rows_v.at[pl.ds(0, 2 * EGRP)],
            acc_sh.at[pl.ds(s * rows_per_tile + i * 2 * EGRP, 2 * EGRP)])
    plsc.subcore_barrier()

    pltpu.sync_copy(srcr_hbm.at[wid, pl.ds(0, CHG)], src_v.at[0])
    pltpu.sync_copy(dstr_hbm.at[wid, pl.ds(0, CHG)], dst_v.at[0])

    def _gather(g):
        slot = (g // CHG) % 2
        pltpu.async_copy(g_hbm.at[src_v.at[slot, g % CHG]],
                         rows_v.at[pl.ds((g % 4) * EGRP, EGRP)], sem)

    _gather(0)
    _gather(1)
    _gather(2)

    def body(g, carry):
        @pl.when(jnp.logical_and(g % CHG == 0, g + CHG < NG_E))
        def _():
            nslot = ((g // CHG) + 1) % 2
            pltpu.sync_copy(
                srcr_hbm.at[wid, pl.ds((g // CHG + 1) * CHG, CHG)],
                src_v.at[nslot])
            pltpu.sync_copy(
                dstr_hbm.at[wid, pl.ds((g // CHG + 1) * CHG, CHG)],
                dst_v.at[nslot])

        slot = (g // CHG) % 2
        buf = (g % 4) * EGRP
        pltpu.make_async_copy(
            g_hbm.at[src_v.at[slot, g % CHG]], rows_v.at[pl.ds(buf, EGRP)],
            sem).wait()

        @pl.when(g + 3 < NG_E)
        def _():
            _gather(g + 3)

        pltpu.sync_copy(rows_v.at[pl.ds(buf, EGRP)],
                        acc_sh.at[dst_v.at[slot, g % CHG]], add=True)
        return carry

    lax.fori_loop(0, NG_E, body, 0)
    plsc.subcore_barrier()
    pltpu.sync_copy(acc_sh.at[pl.ds(s * rows_per_tile, rows_per_tile)],
                    out_hbm.at[c, pl.ds(s * rows_per_tile, rows_per_tile)])


_gcn_call = pl.kernel(
    _gcn_body,
    out_type=jax.ShapeDtypeStruct((2, N_PAD, HID), jnp.float32),
    mesh=_MESH,
    scratch_types=[
        pltpu.VMEM((2, CHG, EGRP), jnp.int32),
        pltpu.VMEM((2, CHG, EGRP), jnp.int32),
        pltpu.VMEM((4 * EGRP, HID), jnp.float32),
        pltpu.VMEM_SHARED((N_PAD, HID), jnp.float32),
        pltpu.SemaphoreType.DMA,
    ],
)


def _mm_body(a_ref, w_ref, o_ref):
    o_ref[...] = jnp.dot(a_ref[...], w_ref[...],
                         preferred_element_type=jnp.float32)


def _mm_fused_body(p0_ref, p1_ref, w_ref, o_ref):
    h = jnp.maximum(p0_ref[0] + p1_ref[0], 0.0)
    o_ref[...] = jnp.dot(h, w_ref[...], preferred_element_type=jnp.float32)


_MM_BLK = N_PAD // 4


def _mm(h, w):
    return pl.pallas_call(
        _mm_body,
        grid=(4,),
        in_specs=[
            pl.BlockSpec((_MM_BLK, HID), lambda i: (i, 0)),
            pl.BlockSpec((HID, HID), lambda i: (0, 0)),
        ],
        out_specs=pl.BlockSpec((_MM_BLK, HID), lambda i: (i, 0)),
        out_shape=jax.ShapeDtypeStruct((N_PAD, HID), jnp.float32),
    )(h, w)


def _mm_fused(p, w):
    return pl.pallas_call(
        _mm_fused_body,
        grid=(4,),
        in_specs=[
            pl.BlockSpec((1, _MM_BLK, HID), lambda i: (0, i, 0)),
            pl.BlockSpec((1, _MM_BLK, HID), lambda i: (1, i, 0)),
            pl.BlockSpec((HID, HID), lambda i: (0, 0)),
        ],
        out_specs=pl.BlockSpec((_MM_BLK, HID), lambda i: (i, 0)),
        out_shape=jax.ShapeDtypeStruct((N_PAD, HID), jnp.float32),
    )(p, p, w)


def kernel(x, edge_index, emb, W0, W1, W2, W3, W4, lin_w):
    xp = jnp.zeros((N_PAD, BAG), jnp.int32).at[:N].set(x)

    src = edge_index[0]
    dst = edge_index[1]
    pad_ids = (jnp.arange(E_PAD - E, dtype=jnp.int32) % (N_PAD - N)) + N
    srcp = jnp.concatenate([src, pad_ids]).reshape(NW, NG_E, EGRP)
    dstp = jnp.concatenate([dst, pad_ids]).reshape(NW, NG_E, EGRP)

    g = _emb_call(xp, emb, W0)
    for W in (W1, W2, W3, W4):
        p = _gcn_call(g, srcp, dstp)
        g = _mm_fused(p, W)
    p = _gcn_call(g, srcp, dstp)

    lin_pad = jnp.zeros((HID, 128), jnp.float32).at[:, :OUT].set(lin_w)
    out = _mm_fused(p, lin_pad)
    return out[:N, :OUT]
